# baseline (speedup 1.0000x reference)
import functools

import jax
import jax.numpy as jnp
from jax import lax
from jax.experimental import pallas as pl
from jax.experimental.pallas import tpu as pltpu
from jax.experimental.pallas import tpu_sc as plsc

EMBED = 1024
VOCAB = 256
PAD = 128
ZROW = VOCAB
NC = 2
NS = 16
NW = NC * NS
LANES = 16

CH = 32


def _body(tbl_hbm, idx_hbm, pos_hbm, out_hbm,
          idx_v, rows_v, pos_v, sem, B, S):
    cid = lax.axis_index("c")
    sid = lax.axis_index("s")
    wid = sid * NC + cid

    n_per_w = S // NW
    n_ch = n_per_w // CH
    s_base = wid * n_per_w

    def chunk_body(ci, _):
        s0 = s_base + ci * CH
        pltpu.sync_copy(pos_hbm.at[pl.ds(s0, CH)], pos_v)

        def batch_body(b, _):
            t0 = b * S + s0
            pltpu.sync_copy(idx_hbm.at[pl.ds(t0, CH)], idx_v)
            for j in range(CH // LANES):
                sl = pl.ds(j * LANES, LANES)
                v = idx_v[sl]
                idx_v[sl] = jnp.where(v == PAD, ZROW, v)
            pltpu.async_copy(tbl_hbm.at[idx_v], rows_v, sem).wait()

            def add_row(r, _):
                for j in range(EMBED // LANES):
                    sl = pl.ds(j * LANES, LANES)
                    rows_v[r, sl] = rows_v[r, sl] + pos_v[r, sl]
                return 0
            lax.fori_loop(0, CH, add_row, 0)

            pltpu.sync_copy(rows_v, out_hbm.at[pl.ds(t0, CH)])
            return 0
        lax.fori_loop(0, B, batch_body, 0)
        return 0
    lax.fori_loop(0, n_ch, chunk_body, 0)


def kernel(inputs, value_table, pos_table):
    B, S = inputs.shape
    idx_flat = inputs.reshape(B * S)
    tbl_pad = jnp.concatenate(
        [value_table, jnp.zeros((8, EMBED), jnp.float32)], axis=0)

    mesh = plsc.VectorSubcoreMesh(
        core_axis_name="c", subcore_axis_name="s",
        num_cores=NC, num_subcores=NS)

    k = functools.partial(
        pl.kernel,
        out_type=jax.ShapeDtypeStruct((B * S, EMBED), jnp.float32),
        mesh=mesh,
        scratch_types=[
            pltpu.VMEM((CH,), jnp.int32),
            pltpu.VMEM((CH, EMBED), jnp.float32),
            pltpu.VMEM((CH, EMBED), jnp.float32),
            pltpu.SemaphoreType.DMA,
        ],
    )(functools.partial(_body, B=B, S=S))

    out = k(tbl_pad, idx_flat, pos_table)
    return out.reshape(B, S, EMBED)
